# baseline (speedup 1.0000x reference)
import functools

import jax
import jax.numpy as jnp
from jax import lax
from jax.experimental import pallas as pl
from jax.experimental.pallas import tpu as pltpu
from jax.experimental.pallas import tpu_sc as plsc

L = 16
T_LEN = 1001
T_PAD = 1008
NVREG = T_PAD // L
NC = 1
NS = 16
NW = NC * NS
NTAB = 6
NTOTV = (NVREG + L - 1) // L


def _rsqrt(x):
    i = plsc.bitcast(x, jnp.int32)
    y = plsc.bitcast(jnp.int32(0x5F3759DF) - (i >> 1), jnp.float32)
    for _ in range(3):
        y = y * (1.5 - 0.5 * x * y * y)
    return y


def _sqrt(x):
    return jnp.where(x > 0.0, x * _rsqrt(x), 0.0)


def _sc_body(t_hbm, bet_hbm, out_hbm, bet_v, lg_v, ps_v, tot_v, ab_v, tab_v,
             t_v, out_v, sem_b, sem_t, sem_o):
    batch = t_hbm.shape[0]
    bpw = batch // NW
    gv = bpw // L
    wid = lax.axis_index("s") * NC + lax.axis_index("c")
    base = wid * bpw

    bet_v[pl.ds(T_PAD - L, L)] = jnp.zeros((L,), jnp.float32)
    cp_b = pltpu.async_copy(bet_hbm, bet_v.at[pl.ds(0, T_LEN)], sem_b)
    cp_t = pltpu.async_copy(t_hbm.at[pl.ds(base, bpw)], t_v, sem_t)
    cp_b.wait()

    iota = lax.iota(jnp.int32, L)

    @plsc.parallel_loop(0, NVREG, unroll=3)
    def p1_body(i):
        s = pl.ds(i * L, L)
        b = bet_v[s]
        p = 0.25 + b * 0.2
        p = 1.0 / 3.0 + b * p
        p = 0.5 + b * p
        l = -b * (1.0 + b * p)
        lg_v[s] = l
        ps_v[s] = plsc.cumsum(l)

    carry = jnp.zeros((L,), jnp.float32)
    for g in range(NTOTV):
        vid = jnp.minimum(g * L + iota, NVREG - 1)
        tg = plsc.load_gather(ps_v, [vid * L + 15])
        sg = plsc.cumsum(tg) + carry
        tot_v[pl.ds(g * L, L)] = sg
        carry = plsc.load_gather(
            tot_v, [jnp.zeros((L,), jnp.int32) + (g * L + 15)]
        )

    @plsc.parallel_loop(0, NVREG, unroll=2)
    def p2_body(i):
        s = i * L
        sl = pl.ds(s, L)
        b = bet_v[sl]
        l = lg_v[sl]
        e_idx = jnp.zeros((L,), jnp.int32) + jnp.maximum(i - 1, 0)
        e = plsc.load_gather(tot_v, [e_idx])
        e = jnp.where(i >= 1, e, 0.0)
        big_s = ps_v[sl] + e
        ab = jnp.exp(big_s)
        ab_v[sl] = ab
        tab_v[pl.ds(0 * T_PAD + s, L)] = _sqrt(b)
        tab_v[pl.ds(1 * T_PAD + s, L)] = jnp.exp(0.5 * big_s)
        tab_v[pl.ds(2 * T_PAD + s, L)] = _sqrt(1.0 - ab)
        tab_v[pl.ds(3 * T_PAD + s, L)] = jnp.exp(-0.5 * l)

    cp_t.wait()

    srcs = (bet_v, tab_v, ab_v, tab_v, tab_v, tab_v)
    offs = (0, 0 * T_PAD, 0, 1 * T_PAD, 2 * T_PAD, 3 * T_PAD)
    copies = []
    for j in range(NTAB):
        src, off = srcs[j], offs[j]

        @plsc.parallel_loop(0, gv, unroll=2)
        def gat_body(i, _src=src, _off=off, _j=j):
            sl = pl.ds(i * L, L)
            idx = t_v[sl]
            out_v[_j, sl] = plsc.load_gather(_src, [idx + _off])

        copies.append(
            pltpu.async_copy(
                out_v.at[j], out_hbm.at[pl.ds(j * batch + base, bpw)], sem_o
            )
        )

    for cp in copies:
        cp.wait()


def _make_sc_call(batch):
    bpw = batch // NW
    mesh = plsc.VectorSubcoreMesh(
        core_axis_name="c", subcore_axis_name="s", num_cores=NC
    )
    return pl.kernel(
        _sc_body,
        mesh=mesh,
        compiler_params=pltpu.CompilerParams(needs_layout_passes=False),
        out_type=jax.ShapeDtypeStruct((NTAB * batch,), jnp.float32),
        scratch_types=[
            pltpu.VMEM((T_PAD,), jnp.float32),
            pltpu.VMEM((T_PAD,), jnp.float32),
            pltpu.VMEM((T_PAD,), jnp.float32),
            pltpu.VMEM((NTOTV * L,), jnp.float32),
            pltpu.VMEM((T_PAD,), jnp.float32),
            pltpu.VMEM((4 * T_PAD,), jnp.float32),
            pltpu.VMEM((bpw,), jnp.int32),
            pltpu.VMEM((NTAB, bpw), jnp.float32),
            pltpu.SemaphoreType.DMA,
            pltpu.SemaphoreType.DMA,
            pltpu.SemaphoreType.DMA,
        ],
    )


@jax.jit
def kernel(t, betas):
    out1d = _make_sc_call(t.shape[0])(t, betas)
    return out1d.reshape(NTAB, -1, 1, 1, 1)


# speedup vs baseline: 1.0358x; 1.0358x over previous
import functools

import jax
import jax.numpy as jnp
from jax import lax
from jax.experimental import pallas as pl
from jax.experimental.pallas import tpu as pltpu
from jax.experimental.pallas import tpu_sc as plsc

L = 16
T_LEN = 1001
T_PAD = 1008
NVREG = T_PAD // L
NC = 1
NS = 16
NW = NC * NS
NTAB = 6
NTOTV = (NVREG + L - 1) // L


def _rsqrt(x):
    i = plsc.bitcast(x, jnp.int32)
    y = plsc.bitcast(jnp.int32(0x5F3759DF) - (i >> 1), jnp.float32)
    for _ in range(3):
        y = y * (1.5 - 0.5 * x * y * y)
    return y


def _sqrt(x):
    return jnp.where(x > 0.0, x * _rsqrt(x), 0.0)


def _sc_body(t_hbm, bet_hbm, out_hbm, bet_v, lg_v, ps_v, tot_v, ab_v, tab_v,
             t_v, out_v, sem_b, sem_t, sem_o):
    batch = t_hbm.shape[0]
    bpw = batch // NW
    gv = bpw // L
    wid = lax.axis_index("s") * NC + lax.axis_index("c")
    base = wid * bpw

    bet_v[pl.ds(T_PAD - L, L)] = jnp.zeros((L,), jnp.float32)
    cp_b = pltpu.async_copy(bet_hbm, bet_v.at[pl.ds(0, T_LEN)], sem_b)
    cp_t = pltpu.async_copy(t_hbm.at[pl.ds(base, bpw)], t_v, sem_t)
    cp_b.wait()

    iota = lax.iota(jnp.int32, L)

    @plsc.parallel_loop(0, NVREG, unroll=3)
    def p1_body(i):
        s = pl.ds(i * L, L)
        b = bet_v[s]
        p = 0.25 + b * 0.2
        p = 1.0 / 3.0 + b * p
        p = 0.5 + b * p
        l = -b * (1.0 + b * p)
        lg_v[s] = l
        ps_v[s] = plsc.cumsum(l)

    carry = jnp.zeros((L,), jnp.float32)
    for g in range(NTOTV):
        vid = jnp.minimum(g * L + iota, NVREG - 1)
        tg = plsc.load_gather(ps_v, [vid * L + 15])
        sg = plsc.cumsum(tg) + carry
        tot_v[pl.ds(g * L, L)] = sg
        carry = plsc.load_gather(
            tot_v, [jnp.zeros((L,), jnp.int32) + (g * L + 15)]
        )

    @plsc.parallel_loop(0, NVREG, unroll=2)
    def p2_body(i):
        s = i * L
        sl = pl.ds(s, L)
        b = bet_v[sl]
        l = lg_v[sl]
        e_idx = jnp.zeros((L,), jnp.int32) + jnp.maximum(i - 1, 0)
        e = plsc.load_gather(tot_v, [e_idx])
        e = jnp.where(i >= 1, e, 0.0)
        big_s = ps_v[sl] + e
        ab = jnp.exp(big_s)
        ab_v[sl] = ab
        tab_v[pl.ds(0 * T_PAD + s, L)] = _sqrt(b)
        tab_v[pl.ds(1 * T_PAD + s, L)] = jnp.exp(0.5 * big_s)
        tab_v[pl.ds(2 * T_PAD + s, L)] = _sqrt(1.0 - ab)
        tab_v[pl.ds(3 * T_PAD + s, L)] = jnp.exp(-0.5 * l)

    cp_t.wait()

    @plsc.parallel_loop(0, gv, unroll=2)
    def gat_body(i):
        sl = pl.ds(i * L, L)
        idx = t_v[sl]
        out_v[0, sl] = plsc.load_gather(bet_v, [idx])
        out_v[1, sl] = plsc.load_gather(tab_v, [idx + 0 * T_PAD])
        out_v[2, sl] = plsc.load_gather(ab_v, [idx])
        out_v[3, sl] = plsc.load_gather(tab_v, [idx + 1 * T_PAD])
        out_v[4, sl] = plsc.load_gather(tab_v, [idx + 2 * T_PAD])
        out_v[5, sl] = plsc.load_gather(tab_v, [idx + 3 * T_PAD])

    copies = [
        pltpu.async_copy(
            out_v.at[j], out_hbm.at[pl.ds(j * batch + base, bpw)], sem_o
        )
        for j in range(NTAB)
    ]
    for cp in copies:
        cp.wait()


def _make_sc_call(batch):
    bpw = batch // NW
    mesh = plsc.VectorSubcoreMesh(
        core_axis_name="c", subcore_axis_name="s", num_cores=NC
    )
    return pl.kernel(
        _sc_body,
        mesh=mesh,
        compiler_params=pltpu.CompilerParams(needs_layout_passes=False),
        out_type=jax.ShapeDtypeStruct((NTAB * batch,), jnp.float32),
        scratch_types=[
            pltpu.VMEM((T_PAD,), jnp.float32),
            pltpu.VMEM((T_PAD,), jnp.float32),
            pltpu.VMEM((T_PAD,), jnp.float32),
            pltpu.VMEM((NTOTV * L,), jnp.float32),
            pltpu.VMEM((T_PAD,), jnp.float32),
            pltpu.VMEM((4 * T_PAD,), jnp.float32),
            pltpu.VMEM((bpw,), jnp.int32),
            pltpu.VMEM((NTAB, bpw), jnp.float32),
            pltpu.SemaphoreType.DMA,
            pltpu.SemaphoreType.DMA,
            pltpu.SemaphoreType.DMA,
        ],
    )


@jax.jit
def kernel(t, betas):
    out1d = _make_sc_call(t.shape[0])(t, betas)
    return out1d.reshape(NTAB, -1, 1, 1, 1)


# speedup vs baseline: 1.0410x; 1.0050x over previous
import functools

import jax
import jax.numpy as jnp
from jax import lax
from jax.experimental import pallas as pl
from jax.experimental.pallas import tpu as pltpu
from jax.experimental.pallas import tpu_sc as plsc

L = 16
T_LEN = 1001
T_PAD = 1008
NVREG = T_PAD // L
NC = 1
NS = 16
NW = NC * NS
NTAB = 6
NTOTV = (NVREG + L - 1) // L


def _rsqrt(x):
    i = plsc.bitcast(x, jnp.int32)
    y = plsc.bitcast(jnp.int32(0x5F3759DF) - (i >> 1), jnp.float32)
    for _ in range(2):
        y = y * (1.5 - 0.5 * x * y * y)
    return y


def _sqrt(x):
    return jnp.where(x > 0.0, x * _rsqrt(x), 0.0)


def _sc_body(t_hbm, bet_hbm, out_hbm, bet_v, lg_v, ps_v, tot_v, ab_v, tab_v,
             t_v, out_v, sem_b, sem_t, sem_o):
    batch = t_hbm.shape[0]
    bpw = batch // NW
    gv = bpw // L
    wid = lax.axis_index("s") * NC + lax.axis_index("c")
    base = wid * bpw

    bet_v[pl.ds(T_PAD - L, L)] = jnp.zeros((L,), jnp.float32)
    cp_b = pltpu.async_copy(bet_hbm, bet_v.at[pl.ds(0, T_LEN)], sem_b)
    cp_t = pltpu.async_copy(t_hbm.at[pl.ds(base, bpw)], t_v, sem_t)
    cp_b.wait()

    iota = lax.iota(jnp.int32, L)

    @plsc.parallel_loop(0, NVREG, unroll=3)
    def p1_body(i):
        s = pl.ds(i * L, L)
        b = bet_v[s]
        p = 0.25 + b * 0.2
        p = 1.0 / 3.0 + b * p
        p = 0.5 + b * p
        l = -b * (1.0 + b * p)
        lg_v[s] = l
        ps_v[s] = plsc.cumsum(l)

    carry = jnp.zeros((L,), jnp.float32)
    for g in range(NTOTV):
        vid = jnp.minimum(g * L + iota, NVREG - 1)
        tg = plsc.load_gather(ps_v, [vid * L + 15])
        sg = plsc.cumsum(tg) + carry
        tot_v[pl.ds(g * L, L)] = sg
        carry = plsc.load_gather(
            tot_v, [jnp.zeros((L,), jnp.int32) + (g * L + 15)]
        )

    @plsc.parallel_loop(0, NVREG, unroll=2)
    def p2_body(i):
        s = i * L
        sl = pl.ds(s, L)
        b = bet_v[sl]
        l = lg_v[sl]
        e_idx = jnp.zeros((L,), jnp.int32) + jnp.maximum(i - 1, 0)
        e = plsc.load_gather(tot_v, [e_idx])
        e = jnp.where(i >= 1, e, 0.0)
        big_s = ps_v[sl] + e
        ab = jnp.exp(big_s)
        ab_v[sl] = ab
        tab_v[pl.ds(0 * T_PAD + s, L)] = _sqrt(b)
        tab_v[pl.ds(1 * T_PAD + s, L)] = jnp.exp(0.5 * big_s)
        tab_v[pl.ds(2 * T_PAD + s, L)] = _sqrt(1.0 - ab)
        tab_v[pl.ds(3 * T_PAD + s, L)] = jnp.exp(-0.5 * l)

    cp_t.wait()

    @plsc.parallel_loop(0, gv, unroll=1)
    def gat_body(i):
        sl = pl.ds(i * L, L)
        idx = t_v[sl]
        out_v[0, sl] = plsc.load_gather(bet_v, [idx])
        out_v[1, sl] = plsc.load_gather(tab_v, [idx + 0 * T_PAD])
        out_v[2, sl] = plsc.load_gather(ab_v, [idx])
        out_v[3, sl] = plsc.load_gather(tab_v, [idx + 1 * T_PAD])
        out_v[4, sl] = plsc.load_gather(tab_v, [idx + 2 * T_PAD])
        out_v[5, sl] = plsc.load_gather(tab_v, [idx + 3 * T_PAD])

    copies = [
        pltpu.async_copy(
            out_v.at[j], out_hbm.at[pl.ds(j * batch + base, bpw)], sem_o
        )
        for j in range(NTAB)
    ]
    for cp in copies:
        cp.wait()


def _make_sc_call(batch):
    bpw = batch // NW
    mesh = plsc.VectorSubcoreMesh(
        core_axis_name="c", subcore_axis_name="s", num_cores=NC
    )
    return pl.kernel(
        _sc_body,
        mesh=mesh,
        compiler_params=pltpu.CompilerParams(needs_layout_passes=False),
        out_type=jax.ShapeDtypeStruct((NTAB * batch,), jnp.float32),
        scratch_types=[
            pltpu.VMEM((T_PAD,), jnp.float32),
            pltpu.VMEM((T_PAD,), jnp.float32),
            pltpu.VMEM((T_PAD,), jnp.float32),
            pltpu.VMEM((NTOTV * L,), jnp.float32),
            pltpu.VMEM((T_PAD,), jnp.float32),
            pltpu.VMEM((4 * T_PAD,), jnp.float32),
            pltpu.VMEM((bpw,), jnp.int32),
            pltpu.VMEM((NTAB, bpw), jnp.float32),
            pltpu.SemaphoreType.DMA,
            pltpu.SemaphoreType.DMA,
            pltpu.SemaphoreType.DMA,
        ],
    )


@jax.jit
def kernel(t, betas):
    out1d = _make_sc_call(t.shape[0])(t, betas)
    return out1d.reshape(NTAB, -1, 1, 1, 1)


# speedup vs baseline: 1.0548x; 1.0133x over previous
import functools

import jax
import jax.numpy as jnp
from jax import lax
from jax.experimental import pallas as pl
from jax.experimental.pallas import tpu as pltpu
from jax.experimental.pallas import tpu_sc as plsc

L = 16
T_LEN = 1001
T_PAD = 1008
NVREG = T_PAD // L
NC = 1
NS = 16
NW = NC * NS
NTAB = 6
NTOTV = (NVREG + L - 1) // L


def _rsqrt(x):
    i = plsc.bitcast(x, jnp.int32)
    y = plsc.bitcast(jnp.int32(0x5F3759DF) - (i >> 1), jnp.float32)
    for _ in range(2):
        y = y * (1.5 - 0.5 * x * y * y)
    return y


def _sqrt(x):
    return jnp.where(x > 0.0, x * _rsqrt(x), 0.0)


def _sc_body(t_hbm, bet_hbm, out_hbm, bet_v, lg_v, ps_v, tot_v, ab_v, tab_v,
             t_v, out_v, sem_b, sem_t, sem_o):
    batch = t_hbm.shape[0]
    bpw = batch // NW
    gv = bpw // L
    wid = lax.axis_index("s") * NC + lax.axis_index("c")
    base = wid * bpw

    bet_v[pl.ds(T_PAD - L, L)] = jnp.zeros((L,), jnp.float32)
    cp_b = pltpu.async_copy(bet_hbm, bet_v.at[pl.ds(0, T_LEN)], sem_b)
    cp_t = pltpu.async_copy(t_hbm.at[pl.ds(base, bpw)], t_v, sem_t)
    cp_b.wait()

    iota = lax.iota(jnp.int32, L)

    @plsc.parallel_loop(0, NVREG, unroll=1)
    def p1_body(i):
        s = pl.ds(i * L, L)
        b = bet_v[s]
        p = 0.25 + b * 0.2
        p = 1.0 / 3.0 + b * p
        p = 0.5 + b * p
        l = -b * (1.0 + b * p)
        lg_v[s] = l
        ps_v[s] = plsc.cumsum(l)

    carry = jnp.zeros((L,), jnp.float32)
    for g in range(NTOTV):
        vid = jnp.minimum(g * L + iota, NVREG - 1)
        tg = plsc.load_gather(ps_v, [vid * L + 15])
        sg = plsc.cumsum(tg) + carry
        tot_v[pl.ds(g * L, L)] = sg
        carry = plsc.load_gather(
            tot_v, [jnp.zeros((L,), jnp.int32) + (g * L + 15)]
        )

    @plsc.parallel_loop(0, NVREG, unroll=1)
    def p2_body(i):
        s = i * L
        sl = pl.ds(s, L)
        b = bet_v[sl]
        l = lg_v[sl]
        e_idx = jnp.zeros((L,), jnp.int32) + jnp.maximum(i - 1, 0)
        e = plsc.load_gather(tot_v, [e_idx])
        e = jnp.where(i >= 1, e, 0.0)
        big_s = ps_v[sl] + e
        ab = jnp.exp(big_s)
        ab_v[sl] = ab
        tab_v[pl.ds(0 * T_PAD + s, L)] = _sqrt(b)
        tab_v[pl.ds(1 * T_PAD + s, L)] = jnp.exp(0.5 * big_s)
        tab_v[pl.ds(2 * T_PAD + s, L)] = _sqrt(1.0 - ab)
        tab_v[pl.ds(3 * T_PAD + s, L)] = jnp.exp(-0.5 * l)

    cp_t.wait()

    @plsc.parallel_loop(0, gv, unroll=1)
    def gat_body(i):
        sl = pl.ds(i * L, L)
        idx = t_v[sl]
        out_v[0, sl] = plsc.load_gather(bet_v, [idx])
        out_v[1, sl] = plsc.load_gather(tab_v, [idx + 0 * T_PAD])
        out_v[2, sl] = plsc.load_gather(ab_v, [idx])
        out_v[3, sl] = plsc.load_gather(tab_v, [idx + 1 * T_PAD])
        out_v[4, sl] = plsc.load_gather(tab_v, [idx + 2 * T_PAD])
        out_v[5, sl] = plsc.load_gather(tab_v, [idx + 3 * T_PAD])

    copies = [
        pltpu.async_copy(
            out_v.at[j], out_hbm.at[pl.ds(j * batch + base, bpw)], sem_o
        )
        for j in range(NTAB)
    ]
    for cp in copies:
        cp.wait()


def _make_sc_call(batch):
    bpw = batch // NW
    mesh = plsc.VectorSubcoreMesh(
        core_axis_name="c", subcore_axis_name="s", num_cores=NC
    )
    return pl.kernel(
        _sc_body,
        mesh=mesh,
        compiler_params=pltpu.CompilerParams(needs_layout_passes=False),
        out_type=jax.ShapeDtypeStruct((NTAB * batch,), jnp.float32),
        scratch_types=[
            pltpu.VMEM((T_PAD,), jnp.float32),
            pltpu.VMEM((T_PAD,), jnp.float32),
            pltpu.VMEM((T_PAD,), jnp.float32),
            pltpu.VMEM((NTOTV * L,), jnp.float32),
            pltpu.VMEM((T_PAD,), jnp.float32),
            pltpu.VMEM((4 * T_PAD,), jnp.float32),
            pltpu.VMEM((bpw,), jnp.int32),
            pltpu.VMEM((NTAB, bpw), jnp.float32),
            pltpu.SemaphoreType.DMA,
            pltpu.SemaphoreType.DMA,
            pltpu.SemaphoreType.DMA,
        ],
    )


@jax.jit
def kernel(t, betas):
    out1d = _make_sc_call(t.shape[0])(t, betas)
    return out1d.reshape(NTAB, -1, 1, 1, 1)


# speedup vs baseline: 1.0634x; 1.0081x over previous
import functools

import jax
import jax.numpy as jnp
from jax import lax
from jax.experimental import pallas as pl
from jax.experimental.pallas import tpu as pltpu
from jax.experimental.pallas import tpu_sc as plsc

L = 16
T_LEN = 1001
T_PAD = 1008
NVREG = T_PAD // L
NC = 1
NS = 16
NW = NC * NS
NTAB = 6
NTOTV = (NVREG + L - 1) // L


def _rsqrt(x):
    i = plsc.bitcast(x, jnp.int32)
    y = plsc.bitcast(jnp.int32(0x5F3759DF) - (i >> 1), jnp.float32)
    for _ in range(2):
        y = y * (1.5 - 0.5 * x * y * y)
    return y


def _sqrt(x):
    return jnp.where(x > 0.0, x * _rsqrt(x), 0.0)


def _sc_body(t_hbm, bet_hbm, out_hbm, bet_v, lg_v, ps_v, tot_v, ab_v, tab_v,
             t_v, out_v, sem_b, sem_t, sem_o):
    batch = t_hbm.shape[0]
    bpw = batch // NW
    gv = bpw // L
    wid = lax.axis_index("s") * NC + lax.axis_index("c")
    base = wid * bpw

    bet_v[pl.ds(T_PAD - L, L)] = jnp.zeros((L,), jnp.float32)
    cp_b = pltpu.async_copy(bet_hbm, bet_v.at[pl.ds(0, T_LEN)], sem_b)
    cp_t = pltpu.async_copy(t_hbm.at[pl.ds(base, bpw)], t_v, sem_t)
    cp_b.wait()

    iota = lax.iota(jnp.int32, L)

    @plsc.parallel_loop(0, NVREG, unroll=1)
    def p1_body(i):
        s = pl.ds(i * L, L)
        b = bet_v[s]
        p = 0.25 + b * 0.2
        p = 1.0 / 3.0 + b * p
        p = 0.5 + b * p
        l = -b * (1.0 + b * p)
        lg_v[s] = l
        ps_v[s] = plsc.cumsum(l)

    @plsc.parallel_loop(0, NTOTV, carry=jnp.zeros((L,), jnp.float32))
    def tot_body(g, carry):
        vid = jnp.minimum(g * L + iota, NVREG - 1)
        tg = plsc.load_gather(ps_v, [vid * L + 15])
        sg = plsc.cumsum(tg) + carry
        tot_v[pl.ds(g * L, L)] = sg
        return plsc.load_gather(
            tot_v, [jnp.zeros((L,), jnp.int32) + (g * L + 15)]
        )

    @plsc.parallel_loop(0, NVREG, unroll=1)
    def p2_body(i):
        s = i * L
        sl = pl.ds(s, L)
        b = bet_v[sl]
        l = lg_v[sl]
        e_idx = jnp.zeros((L,), jnp.int32) + jnp.maximum(i - 1, 0)
        e = plsc.load_gather(tot_v, [e_idx])
        e = jnp.where(i >= 1, e, 0.0)
        big_s = ps_v[sl] + e
        ab = jnp.exp(big_s)
        ab_v[sl] = ab
        tab_v[pl.ds(0 * T_PAD + s, L)] = _sqrt(b)
        tab_v[pl.ds(1 * T_PAD + s, L)] = jnp.exp(0.5 * big_s)
        tab_v[pl.ds(2 * T_PAD + s, L)] = _sqrt(1.0 - ab)
        tab_v[pl.ds(3 * T_PAD + s, L)] = jnp.exp(-0.5 * l)

    cp_t.wait()

    @plsc.parallel_loop(0, gv, unroll=1)
    def gat_body(i):
        s = i * L
        idx = t_v[pl.ds(s, L)]
        out_v[pl.ds(0 * bpw + s, L)] = plsc.load_gather(bet_v, [idx])
        out_v[pl.ds(1 * bpw + s, L)] = plsc.load_gather(tab_v, [idx + 0 * T_PAD])
        out_v[pl.ds(2 * bpw + s, L)] = plsc.load_gather(ab_v, [idx])
        out_v[pl.ds(3 * bpw + s, L)] = plsc.load_gather(tab_v, [idx + 1 * T_PAD])
        out_v[pl.ds(4 * bpw + s, L)] = plsc.load_gather(tab_v, [idx + 2 * T_PAD])
        out_v[pl.ds(5 * bpw + s, L)] = plsc.load_gather(tab_v, [idx + 3 * T_PAD])

    def issue_body(j, _):
        pltpu.make_async_copy(
            out_v.at[pl.ds(j * bpw, bpw)],
            out_hbm.at[pl.ds(j * batch + base, bpw)],
            sem_o,
        ).start()
        return 0

    lax.fori_loop(0, NTAB, issue_body, 0)

    pltpu.make_async_copy(
        out_v, out_hbm.at[pl.ds(base, NTAB * bpw)], sem_o
    ).wait()


def _make_sc_call(batch):
    bpw = batch // NW
    mesh = plsc.VectorSubcoreMesh(
        core_axis_name="c", subcore_axis_name="s", num_cores=NC
    )
    return pl.kernel(
        _sc_body,
        mesh=mesh,
        compiler_params=pltpu.CompilerParams(needs_layout_passes=False),
        out_type=jax.ShapeDtypeStruct((NTAB * batch,), jnp.float32),
        scratch_types=[
            pltpu.VMEM((T_PAD,), jnp.float32),
            pltpu.VMEM((T_PAD,), jnp.float32),
            pltpu.VMEM((T_PAD,), jnp.float32),
            pltpu.VMEM((NTOTV * L,), jnp.float32),
            pltpu.VMEM((T_PAD,), jnp.float32),
            pltpu.VMEM((4 * T_PAD,), jnp.float32),
            pltpu.VMEM((bpw,), jnp.int32),
            pltpu.VMEM((NTAB * bpw,), jnp.float32),
            pltpu.SemaphoreType.DMA,
            pltpu.SemaphoreType.DMA,
            pltpu.SemaphoreType.DMA,
        ],
    )


@jax.jit
def kernel(t, betas):
    out1d = _make_sc_call(t.shape[0])(t, betas)
    return out1d.reshape(NTAB, -1, 1, 1, 1)
